# baseline (speedup 1.0000x reference)
import jax
import jax.numpy as jnp
from jax.experimental import pallas as pl

_DIM = 32
_CODES = 8192
_BLOCK_T = 256


def _vq_block_kernel(x_ref, e_ref, qst_ref, ind_ref, ls_ref):
    x = x_ref[...]
    e = e_ref[...]
    mm = jax.lax.dot_general(x.astype(jnp.bfloat16), e.astype(jnp.bfloat16),
                             (((1,), (1,)), ((), ())),
                             preferred_element_type=jnp.float32)
    xx = jnp.sum(x * x, axis=1, keepdims=True)
    ones = jnp.ones((1, _DIM), dtype=jnp.float32)
    ee = jax.lax.dot_general(ones, e * e, (((1,), (1,)), ((), ())),
                             preferred_element_type=jnp.float32,
                             precision=jax.lax.Precision.HIGHEST)
    dist = -((xx - 2.0 * mm) + ee)
    dmax = jnp.max(dist, axis=1, keepdims=True)
    iota = jax.lax.broadcasted_iota(jnp.int32, (_BLOCK_T, _CODES), 1)
    ind = jnp.min(jnp.where(dist == dmax, iota, _CODES),
                  axis=1, keepdims=True)
    oh = (iota == ind).astype(jnp.float32)
    q = jax.lax.dot_general(oh, e, (((1,), (0,)), ((), ())),
                            preferred_element_type=jnp.float32,
                            precision=jax.lax.Precision.HIGHEST)
    diff = q - x
    qst_ref[...] = x + diff
    ind_ref[...] = ind
    ls_ref[...] = jnp.full((1, 1, 128), jnp.sum(diff * diff), jnp.float32)


def kernel(x, embed):
    B, N, D = x.shape
    flat = x.reshape(-1, D)
    T = flat.shape[0]
    nb = T // _BLOCK_T
    qst, ind, ls = pl.pallas_call(
        _vq_block_kernel,
        grid=(nb,),
        in_specs=[
            pl.BlockSpec((_BLOCK_T, D), lambda i: (i, 0)),
            pl.BlockSpec((_CODES, D), lambda i: (0, 0)),
        ],
        out_specs=[
            pl.BlockSpec((_BLOCK_T, D), lambda i: (i, 0)),
            pl.BlockSpec((_BLOCK_T, 1), lambda i: (i, 0)),
            pl.BlockSpec((1, 1, 128), lambda i: (i, 0, 0)),
        ],
        out_shape=[
            jax.ShapeDtypeStruct((T, D), jnp.float32),
            jax.ShapeDtypeStruct((T, 1), jnp.int32),
            jax.ShapeDtypeStruct((nb, 1, 128), jnp.float32),
        ],
    )(flat, embed)
    quantize_st = qst.reshape(B, N, D)
    embed_ind = ind.reshape(B, N)
    loss = jnp.sum(ls[:, 0, 0]) / (B * N * D)
    return quantize_st, embed_ind, loss


# speedup vs baseline: 1.8296x; 1.8296x over previous
import functools

import jax
import jax.numpy as jnp
from jax import lax
from jax.experimental import pallas as pl
from jax.experimental.pallas import tpu as pltpu
from jax.experimental.pallas import tpu_sc as plsc

_DIM = 32
_CODES = 8192
_BLOCK_T = 256
_TOKENS = 8192


def _argmax_kernel(x_ref, e_ref, ind_ref):
    x = x_ref[...]
    e = e_ref[...]
    mm = jax.lax.dot_general(x, e, (((1,), (1,)), ((), ())),
                             preferred_element_type=jnp.float32)
    xx = jnp.sum(x * x, axis=1, keepdims=True)
    ones = jnp.ones((1, _DIM), dtype=jnp.float32)
    ee = jax.lax.dot_general(ones, e * e, (((1,), (1,)), ((), ())),
                             preferred_element_type=jnp.float32,
                             precision=jax.lax.Precision.HIGHEST)
    dist = -((xx - 2.0 * mm) + ee)
    dmax = jnp.max(dist, axis=1, keepdims=True)
    iota = jax.lax.broadcasted_iota(jnp.int32, (_BLOCK_T, _CODES), 1)
    ind_ref[...] = jnp.min(jnp.where(dist == dmax, iota, _CODES),
                           axis=1, keepdims=True)


def _st_loss_kernel(x_ref, q_ref, qst_ref, ls_ref):
    x = x_ref[...]
    q = q_ref[...]
    diff = q - x
    qst_ref[...] = x + diff
    ls_ref[...] = jnp.full((1, 1, 128), jnp.sum(diff * diff), jnp.float32)


def _make_sc_gather():
    info = plsc.get_sparse_core_info()
    nw = info.num_cores * info.num_subcores
    rows_per_w = _TOKENS // nw
    n_chunks = rows_per_w // 128
    mesh = plsc.VectorSubcoreMesh(core_axis_name="c", subcore_axis_name="s")

    @functools.partial(
        pl.kernel, mesh=mesh,
        out_type=jax.ShapeDtypeStruct((_TOKENS, 128), jnp.float32),
        scratch_types=[
            pltpu.VMEM((n_chunks, 128), jnp.int32),
            pltpu.VMEM((rows_per_w, 128), jnp.float32),
            pltpu.SemaphoreType.DMA,
        ],
    )
    def sc_gather(idx_hbm, table_hbm, out_hbm, idx_v, rows_v, sem):
        wid = lax.axis_index("s") * info.num_cores + lax.axis_index("c")
        base = wid * rows_per_w
        for j in range(n_chunks):
            pltpu.sync_copy(idx_hbm.at[pl.ds(base + j * 128, 128)],
                            idx_v.at[j])
        copies = [
            pltpu.async_copy(table_hbm.at[idx_v.at[j]],
                             rows_v.at[pl.ds(j * 128, 128)], sem)
            for j in range(n_chunks)
        ]
        for c in copies:
            c.wait()
        pltpu.sync_copy(rows_v, out_hbm.at[pl.ds(base, rows_per_w)])

    return sc_gather


def kernel(x, embed):
    B, N, D = x.shape
    flat = x.reshape(-1, D)
    T = flat.shape[0]
    nb = T // _BLOCK_T

    ind = pl.pallas_call(
        _argmax_kernel,
        grid=(nb,),
        in_specs=[
            pl.BlockSpec((_BLOCK_T, D), lambda i: (i, 0)),
            pl.BlockSpec((_CODES, D), lambda i: (0, 0)),
        ],
        out_specs=pl.BlockSpec((_BLOCK_T, 1), lambda i: (i, 0)),
        out_shape=jax.ShapeDtypeStruct((T, 1), jnp.int32),
        compiler_params=pltpu.CompilerParams(
            dimension_semantics=("parallel",)),
    )(flat, embed)

    embed_pad = jnp.pad(embed, ((0, 0), (0, 128 - D)))
    q = _make_sc_gather()(ind.reshape(T), embed_pad)[:, :D]

    qst, ls = pl.pallas_call(
        _st_loss_kernel,
        grid=(nb,),
        in_specs=[
            pl.BlockSpec((_BLOCK_T, D), lambda i: (i, 0)),
            pl.BlockSpec((_BLOCK_T, D), lambda i: (i, 0)),
        ],
        out_specs=[
            pl.BlockSpec((_BLOCK_T, D), lambda i: (i, 0)),
            pl.BlockSpec((1, 1, 128), lambda i: (i, 0, 0)),
        ],
        out_shape=[
            jax.ShapeDtypeStruct((T, D), jnp.float32),
            jax.ShapeDtypeStruct((nb, 1, 128), jnp.float32),
        ],
        compiler_params=pltpu.CompilerParams(
            dimension_semantics=("parallel",)),
    )(flat, q)

    quantize_st = qst.reshape(B, N, D)
    embed_ind = ind.reshape(B, N)
    loss = jnp.sum(ls[:, 0, 0]) / (B * N * D)
    return quantize_st, embed_ind, loss
